# Initial kernel scaffold; baseline (speedup 1.0000x reference)
import jax
import jax.numpy as jnp
from jax.experimental import pallas as pl


def kernel(pred, target):
    raise NotImplementedError("write your pallas kernel here")


# speedup vs baseline: 64.6228x; 64.6228x over previous
import math

import jax
import jax.numpy as jnp
import numpy as np
from jax.experimental import pallas as pl
from jax.experimental.pallas import tpu as pltpu

NLAT = 512
NLON = 1024
LMAX = 50
MMAX = 50
MPAD = 64
ROWS = 8
NPTS = 512


def _cc_weights(n):
    N = n - 1
    theta = np.pi * np.arange(n) / N
    w = np.zeros(n)
    v = np.ones(n - 2)
    if N % 2 == 0:
        w0 = 1.0 / (N * N - 1)
        for k in range(1, N // 2):
            v -= 2.0 * np.cos(2 * k * theta[1:-1]) / (4 * k * k - 1)
        v -= np.cos(N * theta[1:-1]) / (N * N - 1)
    else:
        w0 = 1.0 / (N * N)
        for k in range(1, (N - 1) // 2 + 1):
            v -= 2.0 * np.cos(2 * k * theta[1:-1]) / (4 * k * k - 1)
    w[0] = w0
    w[-1] = w0
    w[1:-1] = 2.0 * v / N
    return w


def _legendre(lmax, mmax, x):
    nx = x.shape[0]
    P = np.zeros((lmax, mmax, nx))
    P[0, 0] = np.sqrt(1.0 / (4.0 * np.pi))
    s = np.sqrt(np.maximum(0.0, 1.0 - x * x))
    for m in range(1, mmax):
        P[m, m] = -np.sqrt((2.0 * m + 1.0) / (2.0 * m)) * s * P[m - 1, m - 1]
    for m in range(0, mmax):
        if m + 1 < lmax:
            P[m + 1, m] = np.sqrt(2.0 * m + 3.0) * x * P[m, m]
        for l in range(m + 2, lmax):
            a = np.sqrt((4.0 * l * l - 1.0) / (l * l - m * m))
            b = np.sqrt(((l - 1.0) ** 2 - m * m) / (4.0 * (l - 1.0) ** 2 - 1.0))
            P[l, m] = a * (x * P[l - 1, m] - b * P[l - 2, m])
    return P


def _build_consts():
    theta = np.pi * np.arange(NLAT) / (NLAT - 1)
    cost = np.cos(theta)
    wq = _cc_weights(NLAT)
    pct = (_legendre(LMAX, MMAX, cost) * wq[None, None, :])
    A = np.zeros((LMAX * MPAD, NLAT), np.float32)
    A.reshape(LMAX, MPAD, NLAT)[:, :MMAX, :] = pct
    n = np.arange(NLON)[:, None]
    m = np.arange(MPAD)[None, :]
    C = np.cos(2.0 * np.pi * m * n / NLON) * (2.0 * np.pi / NLON)
    C[:, MMAX:] = 0.0
    return A.astype(np.float32), C.astype(np.float32)


_A_NP, _C_NP = _build_consts()


def _to_spherical(coords):
    x = coords[..., 0]
    y = coords[..., 1]
    z = coords[..., 2]
    r = jnp.sqrt(x * x + y * y + z * z)
    phi1 = jnp.arccos(jnp.clip(x / r, -1.0, 1.0))
    azn = jnp.sqrt(y * y + z * z)
    a = jnp.arccos(jnp.clip(y / azn, -1.0, 1.0))
    phi2 = a + (2.0 * math.pi - 2.0 * a) * (z < 0) - math.pi
    return r, jnp.stack([phi1, phi2], axis=-1)


def _nn_interp_kernel(pts_ref, r_ref, out_ref):
    cl = pl.program_id(0)
    blk = pl.program_id(1)
    rowi = jax.lax.broadcasted_iota(jnp.int32, (ROWS, NLON), 0).astype(jnp.float32)
    coli = jax.lax.broadcasted_iota(jnp.int32, (ROWS, NLON), 1).astype(jnp.float32)
    scale = math.pi / 512.0
    gx = (rowi + blk.astype(jnp.float32) * ROWS) * scale
    gy = coli * scale - math.pi

    big = jnp.full((ROWS, NLON), 1e30, jnp.float32)
    zero = jnp.zeros((ROWS, NLON), jnp.float32)

    def body(c, carry):
        m1, m2, m3, r1, r2, r3 = carry
        px = pts_ref[cl, c, 0]
        py = pts_ref[cl, c, 1]
        rv = r_ref[cl, c]
        dx = gx - px
        dy = gy - py
        d = dx * dx + dy * dy
        c1 = d < m1
        c2 = d < m2
        c3 = d < m3
        nm3 = jnp.where(c2, m2, jnp.where(c3, d, m3))
        nr3 = jnp.where(c2, r2, jnp.where(c3, rv, r3))
        nm2 = jnp.where(c1, m1, jnp.where(c2, d, m2))
        nr2 = jnp.where(c1, r1, jnp.where(c2, rv, r2))
        nm1 = jnp.where(c1, d, m1)
        nr1 = jnp.where(c1, rv, r1)
        return nm1, nm2, nm3, nr1, nr2, nr3

    m1, m2, m3, r1, r2, r3 = jax.lax.fori_loop(
        0, NPTS, body, (big, big, big, zero, zero, zero), unroll=4)
    s = m1 + m2 + m3
    out_ref[0] = (r1 * m1 + r2 * m2 + r3 * m3) / s


def _sht_loss_kernel(interp_ref, c_ref, a_ref, out_ref):
    diff = interp_ref[0] - interp_ref[1]
    xc = jnp.dot(diff, c_ref[...], preferred_element_type=jnp.float32)
    y = jnp.dot(a_ref[...], xc, preferred_element_type=jnp.float32)
    row_m = jax.lax.broadcasted_iota(jnp.int32, y.shape, 0) % MPAD
    col_m = jax.lax.broadcasted_iota(jnp.int32, y.shape, 1)
    v = jnp.where(row_m == col_m, y, 0.0)
    out_ref[0, 0] = jnp.sum(v * v) / float(LMAX * MMAX)


def kernel(pred, target):
    rp, sp = _to_spherical(pred)
    rt, st = _to_spherical(target)
    pts = jnp.concatenate([sp, st], axis=0)
    rads = jnp.concatenate([rp, rt], axis=0)

    nblk = NLAT // ROWS
    interp = pl.pallas_call(
        _nn_interp_kernel,
        grid=(2, nblk),
        in_specs=[
            pl.BlockSpec(memory_space=pltpu.SMEM),
            pl.BlockSpec(memory_space=pltpu.SMEM),
        ],
        out_specs=pl.BlockSpec((1, ROWS, NLON), lambda cl, b: (cl, b, 0)),
        out_shape=jax.ShapeDtypeStruct((2, NLAT, NLON), jnp.float32),
        compiler_params=pltpu.CompilerParams(
            dimension_semantics=("arbitrary", "arbitrary")),
    )(pts, rads)

    loss = pl.pallas_call(
        _sht_loss_kernel,
        in_specs=[
            pl.BlockSpec(memory_space=pltpu.VMEM),
            pl.BlockSpec(memory_space=pltpu.VMEM),
            pl.BlockSpec(memory_space=pltpu.VMEM),
        ],
        out_specs=pl.BlockSpec(memory_space=pltpu.SMEM),
        out_shape=jax.ShapeDtypeStruct((1, 1), jnp.float32),
    )(interp, jnp.asarray(_C_NP), jnp.asarray(_A_NP))

    return loss[0, 0]


# speedup vs baseline: 229.2992x; 3.5483x over previous
import math

import jax
import jax.numpy as jnp
import numpy as np
from jax.experimental import pallas as pl
from jax.experimental.pallas import tpu as pltpu

NLAT = 512
NLON = 1024
LMAX = 50
MMAX = 50
MPAD = 64
NPTS = 512
TR = 32
TC = 128
NTR = NLAT // TR
NTC = NLON // TC
SCALE = math.pi / 512.0


def _cc_weights(n):
    N = n - 1
    theta = np.pi * np.arange(n) / N
    w = np.zeros(n)
    v = np.ones(n - 2)
    if N % 2 == 0:
        w0 = 1.0 / (N * N - 1)
        for k in range(1, N // 2):
            v -= 2.0 * np.cos(2 * k * theta[1:-1]) / (4 * k * k - 1)
        v -= np.cos(N * theta[1:-1]) / (N * N - 1)
    else:
        w0 = 1.0 / (N * N)
        for k in range(1, (N - 1) // 2 + 1):
            v -= 2.0 * np.cos(2 * k * theta[1:-1]) / (4 * k * k - 1)
    w[0] = w0
    w[-1] = w0
    w[1:-1] = 2.0 * v / N
    return w


def _legendre(lmax, mmax, x):
    nx = x.shape[0]
    P = np.zeros((lmax, mmax, nx))
    P[0, 0] = np.sqrt(1.0 / (4.0 * np.pi))
    s = np.sqrt(np.maximum(0.0, 1.0 - x * x))
    for m in range(1, mmax):
        P[m, m] = -np.sqrt((2.0 * m + 1.0) / (2.0 * m)) * s * P[m - 1, m - 1]
    for m in range(0, mmax):
        if m + 1 < lmax:
            P[m + 1, m] = np.sqrt(2.0 * m + 3.0) * x * P[m, m]
        for l in range(m + 2, lmax):
            a = np.sqrt((4.0 * l * l - 1.0) / (l * l - m * m))
            b = np.sqrt(((l - 1.0) ** 2 - m * m) / (4.0 * (l - 1.0) ** 2 - 1.0))
            P[l, m] = a * (x * P[l - 1, m] - b * P[l - 2, m])
    return P


def _build_consts():
    theta = np.pi * np.arange(NLAT) / (NLAT - 1)
    cost = np.cos(theta)
    wq = _cc_weights(NLAT)
    pct = (_legendre(LMAX, MMAX, cost) * wq[None, None, :])
    A = np.zeros((LMAX * MPAD, NLAT), np.float32)
    A.reshape(LMAX, MPAD, NLAT)[:, :MMAX, :] = pct
    n = np.arange(NLON)[:, None]
    m = np.arange(MPAD)[None, :]
    C = np.cos(2.0 * np.pi * m * n / NLON) * (2.0 * np.pi / NLON)
    C[:, MMAX:] = 0.0
    return A.astype(np.float32), C.astype(np.float32)


_A_NP, _C_NP = _build_consts()

_CX_NP = ((np.arange(NTR) * TR + (TR - 1) / 2.0) * SCALE).astype(np.float32)
_CY_NP = ((np.arange(NTC) * TC + (TC - 1) / 2.0) * SCALE - math.pi).astype(np.float32)
_HX = (TR - 1) / 2.0 * SCALE
_HY = (TC - 1) / 2.0 * SCALE


def _to_spherical(coords):
    x = coords[..., 0]
    y = coords[..., 1]
    z = coords[..., 2]
    r = jnp.sqrt(x * x + y * y + z * z)
    phi1 = jnp.arccos(jnp.clip(x / r, -1.0, 1.0))
    azn = jnp.sqrt(y * y + z * z)
    a = jnp.arccos(jnp.clip(y / azn, -1.0, 1.0))
    phi2 = a + (2.0 * math.pi - 2.0 * a) * (z < 0) - math.pi
    return r, jnp.stack([phi1, phi2], axis=-1)


def _tile_metadata(pts):
    px = pts[:, :, 0][:, None, None, :]
    py = pts[:, :, 1][:, None, None, :]
    cx = jnp.asarray(_CX_NP)[None, :, None, None]
    cy = jnp.asarray(_CY_NP)[None, None, :, None]
    ax = jnp.abs(px - cx)
    ay = jnp.abs(py - cy)
    lbx = jnp.maximum(ax - _HX, 0.0)
    lby = jnp.maximum(ay - _HY, 0.0)
    lb = lbx * lbx + lby * lby
    ubx = ax + _HX
    uby = ay + _HY
    ub = ubx * ubx + uby * uby
    ub3 = -jax.lax.top_k(-ub, 3)[0][..., 2:3]
    keep = lb <= ub3
    key = jnp.where(keep, 0, 1024) + jnp.arange(NPTS, dtype=jnp.int32)
    order = jnp.argsort(key, axis=-1).astype(jnp.int32)
    cnt = jnp.sum(keep, axis=-1, dtype=jnp.int32)
    cnt4 = jnp.minimum((cnt + 3) // 4, NPTS // 4)
    order = order.reshape(2, NTR * NTC, 1, NPTS)
    cnt4 = cnt4.reshape(2, NTR * NTC, 1, 1)
    return order, cnt4


def _nn_interp_kernel(pts_ref, r_ref, order_ref, cnt_ref, out_ref):
    cl = pl.program_id(0)
    tr = pl.program_id(1)
    tc = pl.program_id(2)
    rowi = jax.lax.broadcasted_iota(jnp.int32, (TR, TC), 0).astype(jnp.float32)
    coli = jax.lax.broadcasted_iota(jnp.int32, (TR, TC), 1).astype(jnp.float32)
    gx = (rowi + (tr * TR).astype(jnp.float32)) * SCALE
    gy = (coli + (tc * TC).astype(jnp.float32)) * SCALE - math.pi

    big = jnp.full((TR, TC), 1e30, jnp.float32)
    zero = jnp.zeros((TR, TC), jnp.float32)

    def insert(c, carry):
        m1, m2, m3, r1, r2, r3 = carry
        px = pts_ref[cl, c, 0]
        py = pts_ref[cl, c, 1]
        rv = r_ref[cl, c]
        dx = gx - px
        dy = gy - py
        d = dx * dx + dy * dy
        c1 = d < m1
        c2 = d < m2
        c3 = d < m3
        nm3 = jnp.where(c2, m2, jnp.where(c3, d, m3))
        nr3 = jnp.where(c2, r2, jnp.where(c3, rv, r3))
        nm2 = jnp.where(c1, m1, jnp.where(c2, d, m2))
        nr2 = jnp.where(c1, r1, jnp.where(c2, rv, r2))
        nm1 = jnp.where(c1, d, m1)
        nr1 = jnp.where(c1, rv, r1)
        return nm1, nm2, nm3, nr1, nr2, nr3

    def body(k, carry):
        base = k * 4
        for u in range(4):
            carry = insert(order_ref[0, 0, 0, base + u], carry)
        return carry

    m1, m2, m3, r1, r2, r3 = jax.lax.fori_loop(
        0, cnt_ref[0, 0, 0, 0], body, (big, big, big, zero, zero, zero))
    s = m1 + m2 + m3
    out_ref[0] = (r1 * m1 + r2 * m2 + r3 * m3) / s


def _sht_loss_kernel(interp_ref, c_ref, a_ref, out_ref):
    diff = interp_ref[0] - interp_ref[1]
    xc = jnp.dot(diff, c_ref[...], preferred_element_type=jnp.float32)
    y = jnp.dot(a_ref[...], xc, preferred_element_type=jnp.float32)
    row_m = jax.lax.broadcasted_iota(jnp.int32, y.shape, 0) % MPAD
    col_m = jax.lax.broadcasted_iota(jnp.int32, y.shape, 1)
    v = jnp.where(row_m == col_m, y, 0.0)
    out_ref[0, 0] = jnp.sum(v * v) / float(LMAX * MMAX)


def kernel(pred, target):
    rp, sp = _to_spherical(pred)
    rt, st = _to_spherical(target)
    pts = jnp.concatenate([sp, st], axis=0)
    rads = jnp.concatenate([rp, rt], axis=0)
    order, cnt4 = _tile_metadata(pts)

    interp = pl.pallas_call(
        _nn_interp_kernel,
        grid=(2, NTR, NTC),
        in_specs=[
            pl.BlockSpec(memory_space=pltpu.SMEM),
            pl.BlockSpec(memory_space=pltpu.SMEM),
            pl.BlockSpec((1, 1, 1, NPTS),
                         lambda cl, tr, tc: (cl, tr * NTC + tc, 0, 0),
                         memory_space=pltpu.SMEM),
            pl.BlockSpec((1, 1, 1, 1),
                         lambda cl, tr, tc: (cl, tr * NTC + tc, 0, 0),
                         memory_space=pltpu.SMEM),
        ],
        out_specs=pl.BlockSpec((1, TR, TC), lambda cl, tr, tc: (cl, tr, tc)),
        out_shape=jax.ShapeDtypeStruct((2, NLAT, NLON), jnp.float32),
        compiler_params=pltpu.CompilerParams(
            dimension_semantics=("arbitrary", "arbitrary", "arbitrary")),
    )(pts, rads, order, cnt4)

    loss = pl.pallas_call(
        _sht_loss_kernel,
        in_specs=[
            pl.BlockSpec(memory_space=pltpu.VMEM),
            pl.BlockSpec(memory_space=pltpu.VMEM),
            pl.BlockSpec(memory_space=pltpu.VMEM),
        ],
        out_specs=pl.BlockSpec(memory_space=pltpu.SMEM),
        out_shape=jax.ShapeDtypeStruct((1, 1), jnp.float32),
    )(interp, jnp.asarray(_C_NP), jnp.asarray(_A_NP))

    return loss[0, 0]
